# Initial kernel scaffold; baseline (speedup 1.0000x reference)
import jax
import jax.numpy as jnp
from jax.experimental import pallas as pl


def kernel(x, emb):
    raise NotImplementedError("write your pallas kernel here")


# speedup vs baseline: 2.9465x; 2.9465x over previous
import functools

import jax
import jax.numpy as jnp
from jax import lax
from jax.experimental import pallas as pl
from jax.experimental.pallas import tpu as pltpu
from jax.experimental.pallas import tpu_sc as plsc

NV = 100000
ED = 48
B = 16384
N_CONT = 4
N_ID = 13
X_D = 17
OUT_D = N_CONT + N_ID * ED

NC = 2
NS = 16
NW = NC * NS
ROWS_W = B // NW
CHUNK = 128
N_CHUNK = ROWS_W // CHUNK
UNROLL = 4

_mesh = plsc.VectorSubcoreMesh(
    core_axis_name="c", subcore_axis_name="s", num_cores=NC, num_subcores=NS
)


@functools.partial(
    pl.kernel,
    out_type=jax.ShapeDtypeStruct((B, OUT_D), jnp.float32),
    mesh=_mesh,
    compiler_params=pltpu.CompilerParams(
        needs_layout_passes=False, use_tc_tiling_on_sc=False
    ),
    scratch_types=[
        pltpu.VMEM((CHUNK, X_D), jnp.float32),
        pltpu.VMEM((2, CHUNK), jnp.int32),
        pltpu.VMEM((2, CHUNK, ED), jnp.float32),
        pltpu.VMEM((CHUNK, OUT_D), jnp.float32),
        pltpu.SemaphoreType.DMA,
        pltpu.SemaphoreType.DMA,
    ],
)
def _card_emb(x_hbm, emb_hbm, out_hbm, x_v, idx_v, rows_v, outbuf_v, sem0, sem1):
    wid = lax.axis_index("s") * NC + lax.axis_index("c")
    base = wid * ROWS_W

    lane = lax.iota(jnp.int32, 16)
    sems = (sem0, sem1)

    def chunk_body(k, carry):
        row0 = base + k * CHUNK
        pltpu.sync_copy(x_hbm.at[pl.ds(row0, CHUNK)], x_v)

        def build_and_fire(j):
            buf = j % 2
            idcol = jnp.full((16,), N_CONT + j, jnp.int32)

            def idx_body(g, cc):
                vals = plsc.load_gather(x_v, [g * 16 + lane, idcol])
                idx_v[buf, pl.ds(g * 16, 16)] = vals.astype(jnp.int32)
                return cc

            lax.fori_loop(0, CHUNK // 16, idx_body, 0)
            return pltpu.async_copy(
                emb_hbm.at[idx_v.at[buf]], rows_v.at[buf], sems[buf]
            )

        pending = build_and_fire(0)

        def cont_body(g, c):
            t = lane + g * 16
            row = lax.shift_right_logical(t, 2)
            col = lax.bitwise_and(t, 3)
            vals = plsc.load_gather(x_v, [row, col])
            plsc.store_scatter(outbuf_v, [row, col], vals)
            return c

        lax.fori_loop(0, CHUNK * N_CONT // 16, cont_body, 0)

        for j in range(N_ID):
            nxt = build_and_fire(j + 1) if j + 1 < N_ID else None
            pending.wait()
            pending = nxt
            buf = j % 2
            c0 = N_CONT + j * ED

            def copy_body(i, c, buf=buf, c0=c0):
                r0 = i * UNROLL
                for dr in range(UNROLL):
                    for m in range(ED // 16):
                        v = rows_v[buf, r0 + dr, pl.ds(m * 16, 16)]
                        outbuf_v[r0 + dr, pl.ds(c0 + m * 16, 16)] = v
                return c

            lax.fori_loop(0, CHUNK // UNROLL, copy_body, 0)

        pltpu.sync_copy(outbuf_v, out_hbm.at[pl.ds(row0, CHUNK)])
        return carry

    lax.fori_loop(0, N_CHUNK, chunk_body, 0)


def kernel(x, emb):
    return _card_emb(x, emb)


# speedup vs baseline: 4.9141x; 1.6678x over previous
import functools

import jax
import jax.numpy as jnp
from jax import lax
from jax.experimental import pallas as pl
from jax.experimental.pallas import tpu as pltpu
from jax.experimental.pallas import tpu_sc as plsc

NV = 100000
ED = 48
B = 16384
N_CONT = 4
N_ID = 13
X_D = 17
OUT_D = N_CONT + N_ID * ED

NC = 2
NS = 16
NW = NC * NS
ROWS_W = B // NW
CHUNK = 64
N_CHUNK = ROWS_W // CHUNK
NRING = 4

_mesh = plsc.VectorSubcoreMesh(
    core_axis_name="c", subcore_axis_name="s", num_cores=NC, num_subcores=NS
)


@functools.partial(
    pl.kernel,
    out_type=jax.ShapeDtypeStruct((B, OUT_D), jnp.float32),
    mesh=_mesh,
    compiler_params=pltpu.CompilerParams(
        needs_layout_passes=False, use_tc_tiling_on_sc=False
    ),
    scratch_types=[
        pltpu.VMEM((ROWS_W, X_D), jnp.float32),
        pltpu.VMEM((NRING, CHUNK), jnp.int32),
        pltpu.VMEM((NRING, CHUNK, ED), jnp.float32),
        pltpu.VMEM((2, CHUNK, OUT_D), jnp.float32),
        [pltpu.SemaphoreType.DMA] * NRING,
        [pltpu.SemaphoreType.DMA] * 2,
    ],
)
def _card_emb(x_hbm, emb_hbm, out_hbm, x_v, idx_v, rows_v, outbuf_v, gsems, wsems):
    wid = lax.axis_index("s") * NC + lax.axis_index("c")
    base = wid * ROWS_W

    lane = lax.iota(jnp.int32, 16)

    pltpu.sync_copy(x_hbm.at[pl.ds(base, ROWS_W), :], x_v)

    def chunk_body(k, carry):
        p = lax.bitwise_and(k, 1)
        row0 = base + k * CHUNK
        loc0 = k * CHUNK
        obuf = outbuf_v.at[p]

        for par in range(2):
            @pl.when(jnp.logical_and(k >= 2, p == par))
            def _(par=par):
                pltpu.make_async_copy(
                    obuf, out_hbm.at[pl.ds(base, CHUNK)], wsems[par]
                ).wait()

        def build_and_fire(j):
            slot = j % NRING
            idcol = jnp.full((16,), N_CONT + j, jnp.int32)
            for g in range(CHUNK // 16):
                vals = plsc.load_gather(x_v, [loc0 + g * 16 + lane, idcol])
                idx_v[slot, pl.ds(g * 16, 16)] = vals.astype(jnp.int32)
            pltpu.async_copy(
                emb_hbm.at[idx_v.at[slot]], rows_v.at[slot], gsems[slot]
            )

        for j in range(min(NRING - 1, N_ID)):
            build_and_fire(j)

        @functools.partial(plsc.parallel_loop, 0, CHUNK * N_CONT // 16, unroll=4)
        def _(i):
            t = lane + i * 16
            row = lax.shift_right_logical(t, 2)
            col = lax.bitwise_and(t, 3)
            vals = plsc.load_gather(x_v, [loc0 + row, col])
            plsc.store_scatter(obuf, [row, col], vals)

        for j in range(N_ID):
            if j + NRING - 1 < N_ID:
                build_and_fire(j + NRING - 1)
            slot = j % NRING
            pltpu.make_async_copy(
                emb_hbm.at[idx_v.at[slot]], rows_v.at[slot], gsems[slot]
            ).wait()
            c0 = N_CONT + j * ED
            rbuf = rows_v.at[slot]

            @functools.partial(plsc.parallel_loop, 0, CHUNK, unroll=8)
            def _(r):
                for m in range(ED // 16):
                    obuf[r, pl.ds(c0 + m * 16, 16)] = rbuf[r, pl.ds(m * 16, 16)]

        for par in range(2):
            @pl.when(p == par)
            def _(par=par):
                pltpu.async_copy(
                    obuf, out_hbm.at[pl.ds(row0, CHUNK)], wsems[par]
                )
        return carry

    lax.fori_loop(0, N_CHUNK, chunk_body, 0)

    for par in range(2):
        pltpu.make_async_copy(
            outbuf_v.at[par], out_hbm.at[pl.ds(base, CHUNK)], wsems[par]
        ).wait()


def kernel(x, emb):
    return _card_emb(x, emb)
